# baseline (speedup 1.0000x reference)
import functools

import jax
import jax.numpy as jnp
from jax import lax
from jax.experimental import pallas as pl
from jax.experimental.pallas import tpu as pltpu
from jax.experimental.pallas import tpu_sc as plsc

_N_CLASSES = 1000
_LANES = 16
_ZCHUNK = 2_048_000


def _zero_body(o_ref):
    o_ref[...] = jnp.zeros((_ZCHUNK,), jnp.float32)


@functools.cache
def _make_zero_fill(out_words):
    assert out_words % _ZCHUNK == 0
    return pl.pallas_call(
        _zero_body,
        grid=(out_words // _ZCHUNK,),
        out_specs=pl.BlockSpec((_ZCHUNK,), lambda i: (i,)),
        out_shape=jax.ShapeDtypeStruct((out_words,), jnp.float32),
    )


@functools.cache
def _make_scatter(n_rows, n_classes, seq_len):
    info = plsc.get_sparse_core_info()
    nc, ns = info.num_cores, info.num_subcores
    rows_per_w = n_rows // (nc * ns)
    l_per_sc = seq_len // nc
    b_per_tile = n_rows // seq_len // ns
    magic = (1 << 17) // l_per_sc + 1
    assert all((i * magic) >> 17 == i // l_per_sc for i in range(rows_per_w))
    assert rows_per_w % _LANES == 0
    mesh = plsc.VectorSubcoreMesh(core_axis_name="c", subcore_axis_name="s")

    @functools.partial(
        pl.kernel,
        out_type=(),
        mesh=mesh,
        scratch_types=[
            pltpu.VMEM((b_per_tile, seq_len), jnp.int32),
            pltpu.VMEM((rows_per_w,), jnp.int32),
            pltpu.VMEM((rows_per_w,), jnp.float32),
            pltpu.SemaphoreType.DMA,
        ],
        compiler_params=pltpu.CompilerParams(needs_layout_passes=False),
    )
    def k(x_hbm, out_hbm, x_v, idx_v, ones_v, ssem):
        c = lax.axis_index("c")
        s = lax.axis_index("s")

        ones16 = jnp.ones((_LANES,), jnp.float32)
        iota16 = lax.iota(jnp.int32, _LANES)

        l0 = c * l_per_sc
        b0 = s * b_per_tile
        pltpu.sync_copy(x_hbm.at[pl.ds(b0, b_per_tile)], x_v)

        def idx_body(i, carry):
            flat = i * _LANES + iota16
            bloc = (flat * magic) >> 17
            lloc = flat - bloc * l_per_sc
            b = b0 + bloc
            l = l0 + lloc
            cls = plsc.load_gather(x_v, [bloc, l])
            off = (l * (n_classes * 1024)
                   + (cls >> 3) * 8192
                   + (b >> 7) * 1024
                   + (cls & 7) * 128
                   + (b & 127))
            idx_v[pl.ds(i * _LANES, _LANES)] = off
            ones_v[pl.ds(i * _LANES, _LANES)] = ones16
            return carry

        lax.fori_loop(0, rows_per_w // _LANES, idx_body, 0, unroll=4)

        pltpu.async_copy(ones_v, out_hbm.at[idx_v], ssem).wait()

    return k


def kernel(x):
    b, l = x.shape
    n_rows = b * l
    out_words = n_rows * _N_CLASSES
    buf = jax.new_ref(_make_zero_fill(out_words)())
    _make_scatter(n_rows, _N_CLASSES, l)(x.astype(jnp.int32), buf)
    out1d = buf[...]
    out5 = out1d.reshape(l, _N_CLASSES // 8, b // 128, 8, 128)
    outt = jnp.transpose(out5, (2, 4, 0, 1, 3))
    return outt.reshape(b, l, _N_CLASSES)
